# baseline (speedup 1.0000x reference)
import math

import jax
import jax.numpy as jnp
from jax.experimental import pallas as pl
from jax.experimental.pallas import tpu as pltpu
from jax.experimental.pallas import tpu_sc as plsc

_PAD = 0
_CONF = 0.9
_N = 2048
_V = 32000
_W = 1280
_GRID = _V // _W
_GW = 128
_CHUNKS = _N * _V // _GW
_SC_WINDOW = 128

_L1 = _CONF
_L0 = (1.0 - _CONF) / (_V - 2)
_C = _L1 * math.log(_L1) + (_V - 1) * _L0 * math.log(_L0)


def _sum_body(m_ref, yp_ref, s_ref):
    j = pl.program_id(0)
    part = jnp.sum(yp_ref[...] * m_ref[...])

    @pl.when(j == 0)
    def _():
        s_ref[0, 0] = 0.0

    s_ref[0, 0] += part


def _masked_sum(yp, mrow):
    return pl.pallas_call(
        _sum_body,
        grid=(_GRID,),
        in_specs=[
            pl.BlockSpec((_N, 1), lambda j: (0, 0)),
            pl.BlockSpec((_N, _W), lambda j: (0, j)),
        ],
        out_specs=pl.BlockSpec(memory_space=pltpu.SMEM),
        out_shape=jax.ShapeDtypeStruct((1, 1), jnp.float32),
    )(mrow, yp)


def _sc_gather(chunks2d, idx):
    mesh = plsc.VectorSubcoreMesh(
        core_axis_name="core", subcore_axis_name="subcore"
    )

    @pl.kernel(
        out_type=jax.ShapeDtypeStruct((_N, _GW), jnp.float32), mesh=mesh
    )
    def _k(x_hbm, i_hbm, o_hbm):
        def body(i_vmem, o_vmem):
            pltpu.sync_copy(x_hbm.at[i_vmem.at[0]], o_vmem)

        pltpu.emit_pipeline(
            body,
            grid=(_N // _SC_WINDOW,),
            in_specs=[
                pl.BlockSpec((1, _SC_WINDOW), index_map=lambda i: (0, i))
            ],
            out_specs=[
                pl.BlockSpec((_SC_WINDOW, _GW), index_map=lambda i: (i, 0))
            ],
            core_axis_name="subcore",
            dimension_semantics=(pltpu.PARALLEL,),
        )(i_hbm, o_hbm)

    return _k(chunks2d, idx)


def _combine_body(g_ref, lane_ref, m_ref, s_ref, loss_ref, npad_ref):
    lanes = jax.lax.broadcasted_iota(jnp.int32, (_N, _GW), 1)
    sel = jnp.where(lanes == lane_ref[...], g_ref[...], 0.0)
    gsum = jnp.sum(sel * m_ref[...])
    npad = jnp.sum(m_ref[...])
    npad_ref[0, 0] = npad.astype(jnp.int32)
    loss_ref[0, 0] = npad * _C - _L0 * s_ref[0, 0] - (_L1 - _L0) * gsum


def kernel(y_pred, y_true):
    yp = y_pred.reshape(_N, _V)
    yt = y_true.reshape(_N)
    mrow = (yt != _PAD).astype(jnp.float32).reshape(_N, 1)
    flat = jnp.arange(_N, dtype=jnp.int32) * _V + yt
    chunk = (flat // _GW).reshape(1, _N)
    lane = (flat % _GW).reshape(_N, 1)

    s = _masked_sum(yp, mrow)
    g = _sc_gather(y_pred.reshape(_CHUNKS, _GW), chunk)

    loss, npad = pl.pallas_call(
        _combine_body,
        in_specs=[
            pl.BlockSpec((_N, _GW), lambda: (0, 0)),
            pl.BlockSpec((_N, 1), lambda: (0, 0)),
            pl.BlockSpec((_N, 1), lambda: (0, 0)),
            pl.BlockSpec(memory_space=pltpu.SMEM),
        ],
        out_specs=[
            pl.BlockSpec(memory_space=pltpu.SMEM),
            pl.BlockSpec(memory_space=pltpu.SMEM),
        ],
        out_shape=[
            jax.ShapeDtypeStruct((1, 1), jnp.float32),
            jax.ShapeDtypeStruct((1, 1), jnp.int32),
        ],
    )(g, lane, mrow, s)
    return (loss[0, 0], npad[0, 0])


# speedup vs baseline: 3.0169x; 3.0169x over previous
import math

import jax
import jax.numpy as jnp
from jax.experimental import pallas as pl
from jax.experimental.pallas import tpu as pltpu

_PAD = 0
_CONF = 0.9
_N = 2048
_V = 32000
_W = 1280
_GRID = _V // _W

_L1 = _CONF
_L0 = (1.0 - _CONF) / (_V - 2)
_K = _L1 / _L0
_C = _L1 * math.log(_L1) + (_V - 1) * _L0 * math.log(_L0)


def _body(yts_ref, m_ref, yp_ref, loss_ref, npad_ref):
    j = pl.program_id(0)
    ytj = yts_ref[...] - j * _W
    col = jax.lax.broadcasted_iota(jnp.int32, (_N, _W), 1)
    z = jnp.where(col == ytj, _K, m_ref[...])
    part = jnp.sum(z * yp_ref[...])

    @pl.when(j == 0)
    def _():
        npad_ref[0, 0] = jnp.sum(m_ref[...]).astype(jnp.int32)
        loss_ref[0, 0] = 0.0

    loss_ref[0, 0] += part

    @pl.when(j == _GRID - 1)
    def _():
        loss_ref[0, 0] = (
            npad_ref[0, 0].astype(jnp.float32) * _C - _L0 * loss_ref[0, 0]
        )


def kernel(y_pred, y_true):
    yp = y_pred.reshape(_N, _V)
    yt = y_true.reshape(_N, 1)
    nonpad = yt != _PAD
    yts = jnp.where(nonpad, yt, -1)
    mrow = nonpad.astype(jnp.float32)

    loss, npad = pl.pallas_call(
        _body,
        grid=(_GRID,),
        in_specs=[
            pl.BlockSpec((_N, 1), lambda j: (0, 0)),
            pl.BlockSpec((_N, 1), lambda j: (0, 0)),
            pl.BlockSpec((_N, _W), lambda j: (0, j)),
        ],
        out_specs=[
            pl.BlockSpec(memory_space=pltpu.SMEM),
            pl.BlockSpec(memory_space=pltpu.SMEM),
        ],
        out_shape=[
            jax.ShapeDtypeStruct((1, 1), jnp.float32),
            jax.ShapeDtypeStruct((1, 1), jnp.int32),
        ],
    )(yts, mrow, yp)
    return (loss[0, 0], npad[0, 0])


# speedup vs baseline: 3.5020x; 1.1608x over previous
import math

import jax
import jax.numpy as jnp
from jax.experimental import pallas as pl
from jax.experimental.pallas import tpu as pltpu

_PAD = 0
_CONF = 0.9
_N = 2048
_V = 32000
_W = 1280
_GRID = _V // _W
_SLABS = _W // 128

_L1 = _CONF
_L0 = (1.0 - _CONF) / (_V - 2)
_C = _L1 * math.log(_L1) + (_V - 1) * _L0 * math.log(_L0)


def _body(yts_ref, m_ref, yp_ref, loss_ref, npad_ref, d_ref, s_ref, g_ref):
    j = pl.program_id(0)

    @pl.when(j == 0)
    def _():
        lane = jax.lax.broadcasted_iota(jnp.int32, (_N, 128), 1)
        d_ref[...] = yts_ref[...] - lane
        s_ref[...] = jnp.zeros((_N, 128), jnp.float32)
        g_ref[...] = jnp.zeros((_N, 128), jnp.float32)

    d = d_ref[...]
    part_s = yp_ref[:, 0:128]
    part_g = jnp.where(d == j * _W, part_s, 0.0)
    for c in range(1, _SLABS):
        slab = yp_ref[:, c * 128:(c + 1) * 128]
        part_s = part_s + slab
        part_g = part_g + jnp.where(d == j * _W + c * 128, slab, 0.0)
    s_ref[...] += part_s
    g_ref[...] += part_g

    @pl.when(j == _GRID - 1)
    def _():
        m = m_ref[...]
        npad = jnp.sum(m)
        total_s = jnp.sum(s_ref[...] * m)
        total_g = jnp.sum(g_ref[...])
        npad_ref[0, 0] = npad.astype(jnp.int32)
        loss_ref[0, 0] = npad * _C - _L0 * total_s - (_L1 - _L0) * total_g


def kernel(y_pred, y_true):
    yp = y_pred.reshape(_N, _V)
    yt = y_true.reshape(_N, 1)
    nonpad = yt != _PAD
    yts = jnp.where(nonpad, yt, -1)
    mrow = nonpad.astype(jnp.float32)

    loss, npad = pl.pallas_call(
        _body,
        grid=(_GRID,),
        in_specs=[
            pl.BlockSpec((_N, 1), lambda j: (0, 0)),
            pl.BlockSpec((_N, 1), lambda j: (0, 0)),
            pl.BlockSpec((_N, _W), lambda j: (0, j)),
        ],
        out_specs=[
            pl.BlockSpec(memory_space=pltpu.SMEM),
            pl.BlockSpec(memory_space=pltpu.SMEM),
        ],
        out_shape=[
            jax.ShapeDtypeStruct((1, 1), jnp.float32),
            jax.ShapeDtypeStruct((1, 1), jnp.int32),
        ],
        scratch_shapes=[
            pltpu.VMEM((_N, 128), jnp.int32),
            pltpu.VMEM((_N, 128), jnp.float32),
            pltpu.VMEM((_N, 128), jnp.float32),
        ],
    )(yts, mrow, yp)
    return (loss[0, 0], npad[0, 0])
